# baseline (speedup 1.0000x reference)
import functools

import jax
import jax.numpy as jnp
from jax import lax
from jax.experimental import pallas as pl
from jax.experimental.pallas import tpu as pltpu
from jax.experimental.pallas import tpu_sc as plsc

N = 10000
E = 320000
D = 128
H = 64
C = 40
CP = 48

NC = 2
NS = 16
NW = NC * NS
EPT = E // NW
CH = 125
NCHUNK = EPT // CH
RPT = N // NS
RING = 8
LAG = 4

_SC_MESH = dict(core_axis_name="c", subcore_axis_name="s", num_cores=NC,
                num_subcores=NS)


def _sc_degree(col_r):
    def body(col_hbm, hist_hbm, colidx_v, hist_v):
        cid = lax.axis_index("c")
        sid = lax.axis_index("s")
        wid = cid * NS + sid

        def zrow(i, _):
            hist_v[pl.ds(i * 16, 16)] = jnp.zeros((16,), jnp.float32)
            return 0
        lax.fori_loop(0, N // 16, zrow, 0)

        pltpu.sync_copy(col_hbm.at[wid], colidx_v)
        ones = jnp.ones((16,), jnp.float32)

        def vec(j, _):
            idx = colidx_v[j, :]
            plsc.addupdate_scatter(hist_v, [idx], ones)
            return 0
        lax.fori_loop(0, EPT // 16, vec, 0)

        pltpu.sync_copy(hist_v, hist_hbm.at[wid])

    return pl.kernel(
        body,
        out_type=jax.ShapeDtypeStruct((NW, N), jnp.float32),
        mesh=plsc.VectorSubcoreMesh(**_SC_MESH),
        compiler_params=pltpu.CompilerParams(needs_layout_passes=False),
        scratch_types=[
            pltpu.VMEM((EPT // 16, 16), jnp.int32),
            pltpu.VMEM((N,), jnp.float32),
        ],
    )(col_r)


def _sc_propagate(z, row_r, col_r, feat):
    def body(z_hbm, row_hbm, col_hbm, out_hbm, rowidx_v, colidx_v, rows_v,
             zbuf_v, acc_sh, gsem, ssem):
        cid = lax.axis_index("c")
        sid = lax.axis_index("s")
        wid = cid * NS + sid

        def zrow(i, _):
            def zcol(j, _):
                zbuf_v[i, pl.ds(j * 16, 16)] = jnp.zeros((16,), jnp.float32)
                return 0
            return lax.fori_loop(0, feat // 16, zcol, 0)
        lax.fori_loop(0, 25, zrow, 0)

        def zcopy(k, _):
            pltpu.sync_copy(zbuf_v, acc_sh.at[pl.ds(sid * RPT + k * 25, 25)])
            return 0
        lax.fori_loop(0, RPT // 25, zcopy, 0)

        pltpu.sync_copy(row_hbm.at[wid], rowidx_v)
        pltpu.sync_copy(col_hbm.at[wid], colidx_v)
        plsc.subcore_barrier()

        def issue_gather(j, b):
            pltpu.async_copy(z_hbm.at[rowidx_v.at[j]], rows_v.at[b],
                             gsem.at[b])

        def wait_gather(j, b):
            pltpu.make_async_copy(z_hbm.at[rowidx_v.at[j]], rows_v.at[b],
                                  gsem.at[b]).wait()

        def issue_scatter(j, b):
            pltpu.async_copy(rows_v.at[b], acc_sh.at[colidx_v.at[j]],
                             ssem.at[b], add=True)

        def wait_scatter(j, b):
            pltpu.make_async_copy(rows_v.at[b], acc_sh.at[colidx_v.at[j]],
                                  ssem.at[b]).wait()

        for j in range(LAG):
            issue_gather(j, j % RING)

        def outer(g, _):
            for b in range(RING):
                j = g * RING + b
                wait_gather(j, b)
                issue_scatter(j, b)
                jg = j + LAG
                bg = (b + LAG) % RING

                @pl.when(jg >= RING)
                def _():
                    wait_scatter(jg - RING, bg)

                @pl.when(jg < NCHUNK)
                def _():
                    issue_gather(jg, bg)
            return 0
        lax.fori_loop(0, NCHUNK // RING, outer, 0)

        for j in range(NCHUNK - (RING - LAG), NCHUNK):
            wait_scatter(j, j % RING)

        plsc.subcore_barrier()
        pltpu.sync_copy(acc_sh.at[pl.ds(sid * RPT, RPT)],
                        out_hbm.at[cid, sid])

    return pl.kernel(
        body,
        out_type=jax.ShapeDtypeStruct((NC, NS, RPT, feat), jnp.float32),
        mesh=plsc.VectorSubcoreMesh(**_SC_MESH),
        compiler_params=pltpu.CompilerParams(needs_layout_passes=False,
                                             use_tc_tiling_on_sc=False),
        scratch_types=[
            pltpu.VMEM((NCHUNK, CH), jnp.int32),
            pltpu.VMEM((NCHUNK, CH), jnp.int32),
            pltpu.VMEM((RING, CH, feat), jnp.float32),
            pltpu.VMEM((25, feat), jnp.float32),
            pltpu.VMEM_SHARED((N, feat), jnp.float32),
            pltpu.SemaphoreType.DMA((RING,)),
            pltpu.SemaphoreType.DMA((RING,)),
        ],
    )(z, row_r, col_r)


_BLK = 1000
_GRID = N // _BLK


def _tc_matmul1(x, W1):
    def body(x_ref, w1_ref, z_ref):
        z_ref[...] = jnp.dot(x_ref[...], w1_ref[...],
                             preferred_element_type=jnp.float32)

    return pl.pallas_call(
        body,
        grid=(_GRID,),
        in_specs=[
            pl.BlockSpec((_BLK, D), lambda i: (i, 0)),
            pl.BlockSpec((D, H), lambda i: (0, 0)),
        ],
        out_specs=pl.BlockSpec((_BLK, H), lambda i: (i, 0)),
        out_shape=jax.ShapeDtypeStruct((N, H), jnp.float32),
    )(x, W1)


def _tc_stage1(z1, hist_t):
    def body(z_ref, hist_ref, z1p_ref, dinv_ref):
        deg = 1.0 + jnp.sum(hist_ref[...], axis=1, keepdims=True)
        dinv = lax.rsqrt(deg)
        z1p_ref[...] = z_ref[...] * dinv
        dinv_ref[...] = dinv

    return pl.pallas_call(
        body,
        grid=(_GRID,),
        in_specs=[
            pl.BlockSpec((_BLK, H), lambda i: (i, 0)),
            pl.BlockSpec((_BLK, NW), lambda i: (i, 0)),
        ],
        out_specs=[
            pl.BlockSpec((_BLK, H), lambda i: (i, 0)),
            pl.BlockSpec((_BLK, 1), lambda i: (i, 0)),
        ],
        out_shape=[
            jax.ShapeDtypeStruct((N, H), jnp.float32),
            jax.ShapeDtypeStruct((N, 1), jnp.float32),
        ],
    )(z1, hist_t)


def _tc_stage2(out1, z1p, dinv, b1, W2p):
    def body(p_ref, z1p_ref, dinv_ref, b1_ref, w2_ref, z2p_ref):
        dinv = dinv_ref[...]
        s = p_ref[0] + p_ref[1] + z1p_ref[...]
        h = jnp.maximum(s * dinv + b1_ref[...], 0.0)
        z2 = jnp.dot(h, w2_ref[...], preferred_element_type=jnp.float32)
        z2p_ref[...] = z2 * dinv

    return pl.pallas_call(
        body,
        grid=(_GRID,),
        in_specs=[
            pl.BlockSpec((NC, _BLK, H), lambda i: (0, i, 0)),
            pl.BlockSpec((_BLK, H), lambda i: (i, 0)),
            pl.BlockSpec((_BLK, 1), lambda i: (i, 0)),
            pl.BlockSpec((1, H), lambda i: (0, 0)),
            pl.BlockSpec((H, CP), lambda i: (0, 0)),
        ],
        out_specs=pl.BlockSpec((_BLK, CP), lambda i: (i, 0)),
        out_shape=jax.ShapeDtypeStruct((N, CP), jnp.float32),
    )(out1, z1p, dinv, b1, W2p)


def _tc_stage3(out2, z2p, dinv, b2p):
    def body(p_ref, z2p_ref, dinv_ref, b2_ref, out_ref):
        s = p_ref[0] + p_ref[1] + z2p_ref[...]
        full = s * dinv_ref[...] + b2_ref[...]
        out_ref[...] = full[:, :C]

    return pl.pallas_call(
        body,
        grid=(_GRID,),
        in_specs=[
            pl.BlockSpec((NC, _BLK, CP), lambda i: (0, i, 0)),
            pl.BlockSpec((_BLK, CP), lambda i: (i, 0)),
            pl.BlockSpec((_BLK, 1), lambda i: (i, 0)),
            pl.BlockSpec((1, CP), lambda i: (0, 0)),
        ],
        out_specs=pl.BlockSpec((_BLK, C), lambda i: (i, 0)),
        out_shape=jax.ShapeDtypeStruct((N, C), jnp.float32),
    )(out2, z2p, dinv, b2p)


def kernel(x, edge_index, W1, b1, W2, b2):
    ei = edge_index.astype(jnp.int32)
    row_r = ei[0].reshape(NW, NCHUNK, CH)
    col_r = ei[1].reshape(NW, NCHUNK, CH)
    col_d = ei[1].reshape(NW, EPT // 16, 16)

    z1 = _tc_matmul1(x, W1)
    hist = _sc_degree(col_d)
    hist_t = hist.T

    z1p, dinv = _tc_stage1(z1, hist_t)
    out1 = _sc_propagate(z1p, row_r, col_r, H).reshape(NC, N, H)

    W2p = jnp.pad(W2, ((0, 0), (0, CP - C)))
    b2p = jnp.pad(b2, (0, CP - C)).reshape(1, CP)
    z2p = _tc_stage2(out1, z1p, dinv, b1.reshape(1, H), W2p)

    out2 = _sc_propagate(z2p, row_r, col_r, CP).reshape(NC, N, CP)
    logits = _tc_stage3(out2, z2p, dinv, b2p)
    return logits


# speedup vs baseline: 1.0394x; 1.0394x over previous
import functools

import jax
import jax.numpy as jnp
from jax import lax
from jax.experimental import pallas as pl
from jax.experimental.pallas import tpu as pltpu
from jax.experimental.pallas import tpu_sc as plsc

N = 10000
E = 320000
D = 128
H = 64
C = 40
CP = 48

NC = 2
NS = 16
NW = NC * NS
EPT = E // NW
CH = 125
NCHUNK = EPT // CH
RPT = N // NS
RING = 8
LAG = 4

_SC_MESH = dict(core_axis_name="c", subcore_axis_name="s", num_cores=NC,
                num_subcores=NS)


NP = 10240
_DROWS = NP // 16
_DPT = _DROWS // NS


def _sc_degree(col_d):
    def body(col_hbm, deg_hbm, colidx_v, hist_v, idx_v, acc_sh):
        cid = lax.axis_index("c")
        sid = lax.axis_index("s")
        wid = cid * NS + sid

        zero16 = jnp.zeros((16,), jnp.float32)

        def zrow(i, _):
            hist_v[i, :] = zero16
            return 0
        lax.fori_loop(0, _DROWS, zrow, 0)

        iota = lax.iota(jnp.int32, 16)

        def irow(c, _):
            def icol(i, _):
                idx_v[c, pl.ds(i * 16, 16)] = iota + (c * 128 + i * 16)
                return 0
            return lax.fori_loop(0, 8, icol, 0)
        lax.fori_loop(0, 5, irow, 0)

        pltpu.sync_copy(hist_v.at[pl.ds(0, _DPT)],
                        acc_sh.at[pl.ds(sid * _DPT, _DPT)])

        pltpu.sync_copy(col_hbm.at[wid], colidx_v)
        plsc.subcore_barrier()

        ones = jnp.ones((16,), jnp.float32)

        def vec(j, _):
            idx = colidx_v[j, :]
            plsc.addupdate_scatter(hist_v, [idx >> 4, idx & 15], ones)
            return 0
        lax.fori_loop(0, EPT // 16, vec, 0)

        for c in range(5):
            pltpu.sync_copy(hist_v.at[pl.ds(c * 128, 128)],
                            acc_sh.at[idx_v.at[c]], add=True)
        plsc.subcore_barrier()
        pltpu.sync_copy(acc_sh.at[pl.ds(sid * _DPT, _DPT)],
                        deg_hbm.at[cid, sid])

    return pl.kernel(
        body,
        out_type=jax.ShapeDtypeStruct((NC, NS, _DPT, 16), jnp.float32),
        mesh=plsc.VectorSubcoreMesh(**_SC_MESH),
        compiler_params=pltpu.CompilerParams(needs_layout_passes=False,
                                             use_tc_tiling_on_sc=False),
        scratch_types=[
            pltpu.VMEM((EPT // 16, 16), jnp.int32),
            pltpu.VMEM((_DROWS, 16), jnp.float32),
            pltpu.VMEM((5, 128), jnp.int32),
            pltpu.VMEM_SHARED((_DROWS, 16), jnp.float32),
        ],
    )(col_d)


def _sc_propagate(z, row_r, col_r, feat):
    def body(z_hbm, row_hbm, col_hbm, out_hbm, rowidx_v, colidx_v, rows_v,
             zbuf_v, acc_sh, gsem, ssem):
        cid = lax.axis_index("c")
        sid = lax.axis_index("s")
        wid = cid * NS + sid

        def zrow(i, _):
            def zcol(j, _):
                zbuf_v[i, pl.ds(j * 16, 16)] = jnp.zeros((16,), jnp.float32)
                return 0
            return lax.fori_loop(0, feat // 16, zcol, 0)
        lax.fori_loop(0, 25, zrow, 0)

        def zcopy(k, _):
            pltpu.sync_copy(zbuf_v, acc_sh.at[pl.ds(sid * RPT + k * 25, 25)])
            return 0
        lax.fori_loop(0, RPT // 25, zcopy, 0)

        pltpu.sync_copy(row_hbm.at[wid], rowidx_v)
        pltpu.sync_copy(col_hbm.at[wid], colidx_v)
        plsc.subcore_barrier()

        def issue_gather(j, b):
            pltpu.async_copy(z_hbm.at[rowidx_v.at[j]], rows_v.at[b],
                             gsem.at[b])

        def wait_gather(j, b):
            pltpu.make_async_copy(z_hbm.at[rowidx_v.at[j]], rows_v.at[b],
                                  gsem.at[b]).wait()

        def issue_scatter(j, b):
            pltpu.async_copy(rows_v.at[b], acc_sh.at[colidx_v.at[j]],
                             ssem.at[b], add=True)

        def wait_scatter(j, b):
            pltpu.make_async_copy(rows_v.at[b], acc_sh.at[colidx_v.at[j]],
                                  ssem.at[b]).wait()

        for j in range(LAG):
            issue_gather(j, j % RING)

        def outer(g, _):
            for b in range(RING):
                j = g * RING + b
                wait_gather(j, b)
                issue_scatter(j, b)
                jg = j + LAG
                bg = (b + LAG) % RING

                @pl.when(jg >= RING)
                def _():
                    wait_scatter(jg - RING, bg)

                @pl.when(jg < NCHUNK)
                def _():
                    issue_gather(jg, bg)
            return 0
        lax.fori_loop(0, NCHUNK // RING, outer, 0)

        for j in range(NCHUNK - (RING - LAG), NCHUNK):
            wait_scatter(j, j % RING)

        plsc.subcore_barrier()
        pltpu.sync_copy(acc_sh.at[pl.ds(sid * RPT, RPT)],
                        out_hbm.at[cid, sid])

    return pl.kernel(
        body,
        out_type=jax.ShapeDtypeStruct((NC, NS, RPT, feat), jnp.float32),
        mesh=plsc.VectorSubcoreMesh(**_SC_MESH),
        compiler_params=pltpu.CompilerParams(needs_layout_passes=False,
                                             use_tc_tiling_on_sc=False),
        scratch_types=[
            pltpu.VMEM((NCHUNK, CH), jnp.int32),
            pltpu.VMEM((NCHUNK, CH), jnp.int32),
            pltpu.VMEM((RING, CH, feat), jnp.float32),
            pltpu.VMEM((25, feat), jnp.float32),
            pltpu.VMEM_SHARED((N, feat), jnp.float32),
            pltpu.SemaphoreType.DMA((RING,)),
            pltpu.SemaphoreType.DMA((RING,)),
        ],
    )(z, row_r, col_r)


_BLK = 1000
_GRID = N // _BLK


def _tc_stage1(x, W1, p0, p1):
    def body(x_ref, w1_ref, p0_ref, p1_ref, z1p_ref, dinv_ref):
        deg = 1.0 + p0_ref[...] + p1_ref[...]
        dinv = lax.rsqrt(deg)
        z = jnp.dot(x_ref[...], w1_ref[...],
                    preferred_element_type=jnp.float32)
        z1p_ref[...] = z * dinv
        dinv_ref[...] = dinv

    return pl.pallas_call(
        body,
        grid=(_GRID,),
        in_specs=[
            pl.BlockSpec((_BLK, D), lambda i: (i, 0)),
            pl.BlockSpec((D, H), lambda i: (0, 0)),
            pl.BlockSpec((_BLK, 1), lambda i: (i, 0)),
            pl.BlockSpec((_BLK, 1), lambda i: (i, 0)),
        ],
        out_specs=[
            pl.BlockSpec((_BLK, H), lambda i: (i, 0)),
            pl.BlockSpec((_BLK, 1), lambda i: (i, 0)),
        ],
        out_shape=[
            jax.ShapeDtypeStruct((N, H), jnp.float32),
            jax.ShapeDtypeStruct((N, 1), jnp.float32),
        ],
    )(x, W1, p0, p1)


def _tc_stage2(out1, z1p, dinv, b1, W2p):
    def body(p_ref, z1p_ref, dinv_ref, b1_ref, w2_ref, z2p_ref):
        dinv = dinv_ref[...]
        s = p_ref[0] + p_ref[1] + z1p_ref[...]
        h = jnp.maximum(s * dinv + b1_ref[...], 0.0)
        z2 = jnp.dot(h, w2_ref[...], preferred_element_type=jnp.float32)
        z2p_ref[...] = z2 * dinv

    return pl.pallas_call(
        body,
        grid=(_GRID,),
        in_specs=[
            pl.BlockSpec((NC, _BLK, H), lambda i: (0, i, 0)),
            pl.BlockSpec((_BLK, H), lambda i: (i, 0)),
            pl.BlockSpec((_BLK, 1), lambda i: (i, 0)),
            pl.BlockSpec((1, H), lambda i: (0, 0)),
            pl.BlockSpec((H, CP), lambda i: (0, 0)),
        ],
        out_specs=pl.BlockSpec((_BLK, CP), lambda i: (i, 0)),
        out_shape=jax.ShapeDtypeStruct((N, CP), jnp.float32),
    )(out1, z1p, dinv, b1, W2p)


def _tc_stage3(out2, z2p, dinv, b2p):
    def body(p_ref, z2p_ref, dinv_ref, b2_ref, out_ref):
        s = p_ref[0] + p_ref[1] + z2p_ref[...]
        full = s * dinv_ref[...] + b2_ref[...]
        out_ref[...] = full[:, :C]

    return pl.pallas_call(
        body,
        grid=(_GRID,),
        in_specs=[
            pl.BlockSpec((NC, _BLK, CP), lambda i: (0, i, 0)),
            pl.BlockSpec((_BLK, CP), lambda i: (i, 0)),
            pl.BlockSpec((_BLK, 1), lambda i: (i, 0)),
            pl.BlockSpec((1, CP), lambda i: (0, 0)),
        ],
        out_specs=pl.BlockSpec((_BLK, C), lambda i: (i, 0)),
        out_shape=jax.ShapeDtypeStruct((N, C), jnp.float32),
    )(out2, z2p, dinv, b2p)


def kernel(x, edge_index, W1, b1, W2, b2):
    ei = edge_index.astype(jnp.int32)
    row_r = ei[0].reshape(NW, NCHUNK, CH)
    col_r = ei[1].reshape(NW, NCHUNK, CH)
    col_d = ei[1].reshape(NW, EPT // 16, 16)

    degp = _sc_degree(col_d).reshape(NC, NP, 1)

    z1p, dinv = _tc_stage1(x, W1, degp[0], degp[1])
    out1 = _sc_propagate(z1p, row_r, col_r, H).reshape(NC, N, H)

    W2p = jnp.pad(W2, ((0, 0), (0, CP - C)))
    b2p = jnp.pad(b2, (0, CP - C)).reshape(1, CP)
    z2p = _tc_stage2(out1, z1p, dinv, b1.reshape(1, H), W2p)

    out2 = _sc_propagate(z2p, row_r, col_r, CP).reshape(NC, N, CP)
    logits = _tc_stage3(out2, z2p, dinv, b2p)
    return logits


# speedup vs baseline: 1.2183x; 1.1722x over previous
import functools

import jax
import jax.numpy as jnp
from jax import lax
from jax.experimental import pallas as pl
from jax.experimental.pallas import tpu as pltpu
from jax.experimental.pallas import tpu_sc as plsc

N = 10000
E = 320000
D = 128
H = 64
C = 40
CP = 64

NC = 2
NS = 16
NW = NC * NS
EPT = E // NW
CH = 125
NCHUNK = EPT // CH
RPT = N // NS
RING = 8
LAG = 4

_SC_MESH = dict(core_axis_name="c", subcore_axis_name="s", num_cores=NC,
                num_subcores=NS)


NP = 10240
_DROWS = NP // 16
_DPT = _DROWS // NS


def _sc_degree(col_d):
    def body(col_hbm, deg_hbm, colidx_v, hist_v, idx_v, acc_sh):
        cid = lax.axis_index("c")
        sid = lax.axis_index("s")
        wid = cid * NS + sid

        zero16 = jnp.zeros((16,), jnp.float32)

        def zrow(i, _):
            hist_v[i, :] = zero16
            return 0
        lax.fori_loop(0, _DROWS, zrow, 0)

        iota = lax.iota(jnp.int32, 16)

        def irow(c, _):
            def icol(i, _):
                idx_v[c, pl.ds(i * 16, 16)] = iota + (c * 128 + i * 16)
                return 0
            return lax.fori_loop(0, 8, icol, 0)
        lax.fori_loop(0, 5, irow, 0)

        pltpu.sync_copy(hist_v.at[pl.ds(0, _DPT)],
                        acc_sh.at[pl.ds(sid * _DPT, _DPT)])

        pltpu.sync_copy(col_hbm.at[wid], colidx_v)
        plsc.subcore_barrier()

        ones = jnp.ones((16,), jnp.float32)

        def vec(j, _):
            idx = colidx_v[j, :]
            plsc.addupdate_scatter(hist_v, [idx >> 4, idx & 15], ones)
            return 0
        lax.fori_loop(0, EPT // 16, vec, 0)

        for c in range(5):
            pltpu.sync_copy(hist_v.at[pl.ds(c * 128, 128)],
                            acc_sh.at[idx_v.at[c]], add=True)
        plsc.subcore_barrier()
        pltpu.sync_copy(acc_sh.at[pl.ds(sid * _DPT, _DPT)],
                        deg_hbm.at[cid, sid])

    return pl.kernel(
        body,
        out_type=jax.ShapeDtypeStruct((NC, NS, _DPT, 16), jnp.float32),
        mesh=plsc.VectorSubcoreMesh(**_SC_MESH),
        compiler_params=pltpu.CompilerParams(needs_layout_passes=False,
                                             use_tc_tiling_on_sc=False),
        scratch_types=[
            pltpu.VMEM((EPT // 16, 16), jnp.int32),
            pltpu.VMEM((_DROWS, 16), jnp.float32),
            pltpu.VMEM((5, 128), jnp.int32),
            pltpu.VMEM_SHARED((_DROWS, 16), jnp.float32),
        ],
    )(col_d)


def _sc_propagate(z, row_r, col_r, feat):
    def body(z_hbm, row_hbm, col_hbm, out_hbm, rowidx_v, colidx_v, rows_v,
             zbuf_v, acc_sh, gsem, ssem):
        cid = lax.axis_index("c")
        sid = lax.axis_index("s")
        wid = cid * NS + sid

        rowcp = pltpu.async_copy(row_hbm.at[wid], rowidx_v, gsem.at[0])
        colcp = pltpu.async_copy(col_hbm.at[wid], colidx_v, gsem.at[1])

        def zrow(i, _):
            def zcol(j, _):
                zbuf_v[i, pl.ds(j * 32, 32)] = jnp.zeros((32,), jnp.bfloat16)
                return 0
            return lax.fori_loop(0, feat // 32, zcol, 0)
        lax.fori_loop(0, 25, zrow, 0)

        def zcopy(k, _):
            pltpu.sync_copy(zbuf_v, acc_sh.at[pl.ds(sid * RPT + k * 25, 25)])
            return 0
        lax.fori_loop(0, RPT // 25, zcopy, 0)

        rowcp.wait()
        colcp.wait()
        plsc.subcore_barrier()

        def issue_gather(j, b):
            pltpu.async_copy(z_hbm.at[rowidx_v.at[j]], rows_v.at[b],
                             gsem.at[b])

        def wait_gather(j, b):
            pltpu.make_async_copy(z_hbm.at[rowidx_v.at[j]], rows_v.at[b],
                                  gsem.at[b]).wait()

        def issue_scatter(j, b):
            pltpu.async_copy(rows_v.at[b], acc_sh.at[colidx_v.at[j]],
                             ssem.at[b], add=True)

        def wait_scatter(j, b):
            pltpu.make_async_copy(rows_v.at[b], acc_sh.at[colidx_v.at[j]],
                                  ssem.at[b]).wait()

        for j in range(LAG):
            issue_gather(j, j % RING)

        def outer(g, _):
            for b in range(RING):
                j = g * RING + b
                wait_gather(j, b)
                issue_scatter(j, b)
                jg = j + LAG
                bg = (b + LAG) % RING

                @pl.when(jg >= RING)
                def _():
                    wait_scatter(jg - RING, bg)

                @pl.when(jg < NCHUNK)
                def _():
                    issue_gather(jg, bg)
            return 0
        lax.fori_loop(0, NCHUNK // RING, outer, 0)

        for j in range(NCHUNK - (RING - LAG), NCHUNK):
            wait_scatter(j, j % RING)

        plsc.subcore_barrier()
        pltpu.sync_copy(acc_sh.at[pl.ds(sid * RPT, RPT)],
                        out_hbm.at[cid, sid])

    return pl.kernel(
        body,
        out_type=jax.ShapeDtypeStruct((NC, NS, RPT, feat), jnp.bfloat16),
        mesh=plsc.VectorSubcoreMesh(**_SC_MESH),
        compiler_params=pltpu.CompilerParams(needs_layout_passes=False,
                                             use_tc_tiling_on_sc=False),
        scratch_types=[
            pltpu.VMEM((NCHUNK, CH), jnp.int32),
            pltpu.VMEM((NCHUNK, CH), jnp.int32),
            pltpu.VMEM((RING, CH, feat), jnp.bfloat16),
            pltpu.VMEM((25, feat), jnp.bfloat16),
            pltpu.VMEM_SHARED((N, feat), jnp.bfloat16),
            pltpu.SemaphoreType.DMA((RING,)),
            pltpu.SemaphoreType.DMA((RING,)),
        ],
    )(z, row_r, col_r)


_BLK = 1000
_GRID = N // _BLK


def _tc_stage1(x, W1, degp):
    def body(x_ref, w1_ref, p_ref, z1p_ref, dinv_ref):
        deg = 1.0 + p_ref[0] + p_ref[1]
        dinv = lax.rsqrt(deg)
        z = jnp.dot(x_ref[...], w1_ref[...],
                    preferred_element_type=jnp.float32)
        z1p_ref[...] = (z * dinv).astype(jnp.bfloat16)
        dinv_ref[...] = dinv

    return pl.pallas_call(
        body,
        grid=(_GRID,),
        in_specs=[
            pl.BlockSpec((_BLK, D), lambda i: (i, 0)),
            pl.BlockSpec((D, H), lambda i: (0, 0)),
            pl.BlockSpec((NC, _BLK, 1), lambda i: (0, i, 0)),
        ],
        out_specs=[
            pl.BlockSpec((_BLK, H), lambda i: (i, 0)),
            pl.BlockSpec((_BLK, 1), lambda i: (i, 0)),
        ],
        out_shape=[
            jax.ShapeDtypeStruct((N, H), jnp.bfloat16),
            jax.ShapeDtypeStruct((N, 1), jnp.float32),
        ],
    )(x, W1, degp)


def _tc_stage2(out1, z1p, dinv, b1, W2p):
    def body(p_ref, z1p_ref, dinv_ref, b1_ref, w2_ref, z2p_ref):
        dinv = dinv_ref[...]
        s = (p_ref[0].astype(jnp.float32) + p_ref[1].astype(jnp.float32)
             + z1p_ref[...].astype(jnp.float32))
        h = jnp.maximum(s * dinv + b1_ref[...], 0.0)
        z2 = jnp.dot(h, w2_ref[...], preferred_element_type=jnp.float32)
        z2p_ref[...] = (z2 * dinv).astype(jnp.bfloat16)

    return pl.pallas_call(
        body,
        grid=(_GRID,),
        in_specs=[
            pl.BlockSpec((NC, _BLK, H), lambda i: (0, i, 0)),
            pl.BlockSpec((_BLK, H), lambda i: (i, 0)),
            pl.BlockSpec((_BLK, 1), lambda i: (i, 0)),
            pl.BlockSpec((1, H), lambda i: (0, 0)),
            pl.BlockSpec((H, CP), lambda i: (0, 0)),
        ],
        out_specs=pl.BlockSpec((_BLK, CP), lambda i: (i, 0)),
        out_shape=jax.ShapeDtypeStruct((N, CP), jnp.bfloat16),
    )(out1, z1p, dinv, b1, W2p)


def _tc_stage3(out2, z2p, dinv, b2p):
    def body(p_ref, z2p_ref, dinv_ref, b2_ref, out_ref):
        s = (p_ref[0].astype(jnp.float32) + p_ref[1].astype(jnp.float32)
             + z2p_ref[...].astype(jnp.float32))
        full = s * dinv_ref[...] + b2_ref[...]
        out_ref[...] = full[:, :C]

    return pl.pallas_call(
        body,
        grid=(_GRID,),
        in_specs=[
            pl.BlockSpec((NC, _BLK, CP), lambda i: (0, i, 0)),
            pl.BlockSpec((_BLK, CP), lambda i: (i, 0)),
            pl.BlockSpec((_BLK, 1), lambda i: (i, 0)),
            pl.BlockSpec((1, CP), lambda i: (0, 0)),
        ],
        out_specs=pl.BlockSpec((_BLK, C), lambda i: (i, 0)),
        out_shape=jax.ShapeDtypeStruct((N, C), jnp.float32),
    )(out2, z2p, dinv, b2p)


def kernel(x, edge_index, W1, b1, W2, b2):
    ei = edge_index.astype(jnp.int32)
    row_r = ei[0].reshape(NW, NCHUNK, CH)
    col_r = ei[1].reshape(NW, NCHUNK, CH)
    col_d = ei[1].reshape(NW, EPT // 16, 16)

    degp = _sc_degree(col_d).reshape(NC, NP, 1)

    z1p, dinv = _tc_stage1(x, W1, degp)
    out1 = _sc_propagate(z1p, row_r, col_r, H).reshape(NC, N, H)

    W2p = jnp.pad(W2, ((0, 0), (0, CP - C)))
    b2p = jnp.pad(b2, (0, CP - C)).reshape(1, CP)
    z2p = _tc_stage2(out1, z1p, dinv, b1.reshape(1, H), W2p)

    out2 = _sc_propagate(z2p, row_r, col_r, CP).reshape(NC, N, CP)
    logits = _tc_stage3(out2, z2p, dinv, b2p)
    return logits


# speedup vs baseline: 1.2636x; 1.0371x over previous
import functools

import jax
import jax.numpy as jnp
from jax import lax
from jax.experimental import pallas as pl
from jax.experimental.pallas import tpu as pltpu
from jax.experimental.pallas import tpu_sc as plsc

N = 10000
E = 320000
D = 128
H = 64
C = 40
CP = 64

NC = 2
NS = 16
NW = NC * NS
EPT = E // NW
CH = 125
NCHUNK = EPT // CH
RPT = N // NS
RING = 8
LAG = 6

_SC_MESH = dict(core_axis_name="c", subcore_axis_name="s", num_cores=NC,
                num_subcores=NS)


NP = 10240
_DROWS = NP // 16
_DPT = _DROWS // NS


def _sc_degree(col_d):
    def body(col_hbm, deg_hbm, colidx_v, hist_v, idx_v, acc_sh):
        cid = lax.axis_index("c")
        sid = lax.axis_index("s")
        wid = cid * NS + sid

        zero16 = jnp.zeros((16,), jnp.float32)

        def zrow(i, _):
            hist_v[i, :] = zero16
            return 0
        lax.fori_loop(0, _DROWS, zrow, 0)

        iota = lax.iota(jnp.int32, 16)

        def irow(c, _):
            def icol(i, _):
                idx_v[c, pl.ds(i * 16, 16)] = iota + (c * 128 + i * 16)
                return 0
            return lax.fori_loop(0, 8, icol, 0)
        lax.fori_loop(0, 5, irow, 0)

        pltpu.sync_copy(hist_v.at[pl.ds(0, _DPT)],
                        acc_sh.at[pl.ds(sid * _DPT, _DPT)])

        pltpu.sync_copy(col_hbm.at[wid], colidx_v)
        plsc.subcore_barrier()

        ones = jnp.ones((16,), jnp.float32)

        def vec(j, _):
            idx = colidx_v[j, :]
            plsc.addupdate_scatter(hist_v, [idx >> 4, idx & 15], ones)
            return 0
        lax.fori_loop(0, EPT // 16, vec, 0)

        for c in range(5):
            pltpu.sync_copy(hist_v.at[pl.ds(c * 128, 128)],
                            acc_sh.at[idx_v.at[c]], add=True)
        plsc.subcore_barrier()
        pltpu.sync_copy(acc_sh.at[pl.ds(sid * _DPT, _DPT)],
                        deg_hbm.at[cid, sid])

    return pl.kernel(
        body,
        out_type=jax.ShapeDtypeStruct((NC, NS, _DPT, 16), jnp.float32),
        mesh=plsc.VectorSubcoreMesh(**_SC_MESH),
        compiler_params=pltpu.CompilerParams(needs_layout_passes=False,
                                             use_tc_tiling_on_sc=False),
        scratch_types=[
            pltpu.VMEM((EPT // 16, 16), jnp.int32),
            pltpu.VMEM((_DROWS, 16), jnp.float32),
            pltpu.VMEM((5, 128), jnp.int32),
            pltpu.VMEM_SHARED((_DROWS, 16), jnp.float32),
        ],
    )(col_d)


def _sc_propagate(z, row_r, col_r, feat):
    def body(z_hbm, row_hbm, col_hbm, out_hbm, rowidx_v, colidx_v, rows_v,
             zbuf_v, acc_sh, gsem, ssem):
        cid = lax.axis_index("c")
        sid = lax.axis_index("s")
        wid = cid * NS + sid

        rowcp = pltpu.async_copy(row_hbm.at[wid], rowidx_v, gsem.at[0])
        colcp = pltpu.async_copy(col_hbm.at[wid], colidx_v, gsem.at[1])

        def zrow(i, _):
            def zcol(j, _):
                zbuf_v[i, pl.ds(j * 32, 32)] = jnp.zeros((32,), jnp.bfloat16)
                return 0
            return lax.fori_loop(0, feat // 32, zcol, 0)
        lax.fori_loop(0, 25, zrow, 0)

        def zcopy(k, _):
            pltpu.sync_copy(zbuf_v, acc_sh.at[pl.ds(sid * RPT + k * 25, 25)])
            return 0
        lax.fori_loop(0, RPT // 25, zcopy, 0)

        rowcp.wait()
        colcp.wait()
        plsc.subcore_barrier()

        def issue_gather(j, b):
            pltpu.async_copy(z_hbm.at[rowidx_v.at[j]], rows_v.at[b],
                             gsem.at[b])

        def wait_gather(j, b):
            pltpu.make_async_copy(z_hbm.at[rowidx_v.at[j]], rows_v.at[b],
                                  gsem.at[b]).wait()

        def issue_scatter(j, b):
            pltpu.async_copy(rows_v.at[b], acc_sh.at[colidx_v.at[j]],
                             ssem.at[b], add=True)

        def wait_scatter(j, b):
            pltpu.make_async_copy(rows_v.at[b], acc_sh.at[colidx_v.at[j]],
                                  ssem.at[b]).wait()

        for j in range(LAG):
            issue_gather(j, j % RING)

        def outer(g, _):
            for b in range(RING):
                j = g * RING + b
                wait_gather(j, b)
                issue_scatter(j, b)
                jg = j + LAG
                bg = (b + LAG) % RING

                @pl.when(jg >= RING)
                def _():
                    wait_scatter(jg - RING, bg)

                @pl.when(jg < NCHUNK)
                def _():
                    issue_gather(jg, bg)
            return 0
        lax.fori_loop(0, NCHUNK // RING, outer, 0)

        for j in range(NCHUNK - (RING - LAG), NCHUNK):
            wait_scatter(j, j % RING)

        plsc.subcore_barrier()
        pltpu.sync_copy(acc_sh.at[pl.ds(sid * RPT, RPT)],
                        out_hbm.at[cid, sid])

    return pl.kernel(
        body,
        out_type=jax.ShapeDtypeStruct((NC, NS, RPT, feat), jnp.bfloat16),
        mesh=plsc.VectorSubcoreMesh(**_SC_MESH),
        compiler_params=pltpu.CompilerParams(needs_layout_passes=False,
                                             use_tc_tiling_on_sc=False),
        scratch_types=[
            pltpu.VMEM((NCHUNK, CH), jnp.int32),
            pltpu.VMEM((NCHUNK, CH), jnp.int32),
            pltpu.VMEM((RING, CH, feat), jnp.bfloat16),
            pltpu.VMEM((25, feat), jnp.bfloat16),
            pltpu.VMEM_SHARED((N, feat), jnp.bfloat16),
            pltpu.SemaphoreType.DMA((RING,)),
            pltpu.SemaphoreType.DMA((RING,)),
        ],
    )(z, row_r, col_r)


_BLK = 1000
_GRID = N // _BLK


def _tc_stage1(x, W1, degp):
    def body(x_ref, w1_ref, p_ref, z1p_ref, dinv_ref):
        deg = 1.0 + p_ref[0] + p_ref[1]
        dinv = lax.rsqrt(deg)
        z = jnp.dot(x_ref[...], w1_ref[...],
                    preferred_element_type=jnp.float32)
        z1p_ref[...] = (z * dinv).astype(jnp.bfloat16)
        dinv_ref[...] = dinv

    return pl.pallas_call(
        body,
        grid=(_GRID,),
        in_specs=[
            pl.BlockSpec((_BLK, D), lambda i: (i, 0)),
            pl.BlockSpec((D, H), lambda i: (0, 0)),
            pl.BlockSpec((NC, _BLK, 1), lambda i: (0, i, 0)),
        ],
        out_specs=[
            pl.BlockSpec((_BLK, H), lambda i: (i, 0)),
            pl.BlockSpec((_BLK, 1), lambda i: (i, 0)),
        ],
        out_shape=[
            jax.ShapeDtypeStruct((N, H), jnp.bfloat16),
            jax.ShapeDtypeStruct((N, 1), jnp.float32),
        ],
    )(x, W1, degp)


def _tc_stage2(out1, z1p, dinv, b1, W2p):
    def body(p_ref, z1p_ref, dinv_ref, b1_ref, w2_ref, z2p_ref):
        dinv = dinv_ref[...]
        s = (p_ref[0].astype(jnp.float32) + p_ref[1].astype(jnp.float32)
             + z1p_ref[...].astype(jnp.float32))
        h = jnp.maximum(s * dinv + b1_ref[...], 0.0)
        z2 = jnp.dot(h, w2_ref[...], preferred_element_type=jnp.float32)
        z2p_ref[...] = (z2 * dinv).astype(jnp.bfloat16)

    return pl.pallas_call(
        body,
        grid=(_GRID,),
        in_specs=[
            pl.BlockSpec((NC, _BLK, H), lambda i: (0, i, 0)),
            pl.BlockSpec((_BLK, H), lambda i: (i, 0)),
            pl.BlockSpec((_BLK, 1), lambda i: (i, 0)),
            pl.BlockSpec((1, H), lambda i: (0, 0)),
            pl.BlockSpec((H, CP), lambda i: (0, 0)),
        ],
        out_specs=pl.BlockSpec((_BLK, CP), lambda i: (i, 0)),
        out_shape=jax.ShapeDtypeStruct((N, CP), jnp.bfloat16),
    )(out1, z1p, dinv, b1, W2p)


def _tc_stage3(out2, z2p, dinv, b2p):
    def body(p_ref, z2p_ref, dinv_ref, b2_ref, out_ref):
        s = (p_ref[0].astype(jnp.float32) + p_ref[1].astype(jnp.float32)
             + z2p_ref[...].astype(jnp.float32))
        full = s * dinv_ref[...] + b2_ref[...]
        out_ref[...] = full[:, :C]

    return pl.pallas_call(
        body,
        grid=(_GRID,),
        in_specs=[
            pl.BlockSpec((NC, _BLK, CP), lambda i: (0, i, 0)),
            pl.BlockSpec((_BLK, CP), lambda i: (i, 0)),
            pl.BlockSpec((_BLK, 1), lambda i: (i, 0)),
            pl.BlockSpec((1, CP), lambda i: (0, 0)),
        ],
        out_specs=pl.BlockSpec((_BLK, C), lambda i: (i, 0)),
        out_shape=jax.ShapeDtypeStruct((N, C), jnp.float32),
    )(out2, z2p, dinv, b2p)


def kernel(x, edge_index, W1, b1, W2, b2):
    ei = edge_index.astype(jnp.int32)
    row_r = ei[0].reshape(NW, NCHUNK, CH)
    col_r = ei[1].reshape(NW, NCHUNK, CH)
    col_d = ei[1].reshape(NW, EPT // 16, 16)

    degp = _sc_degree(col_d).reshape(NC, NP, 1)

    z1p, dinv = _tc_stage1(x, W1, degp)
    out1 = _sc_propagate(z1p, row_r, col_r, H).reshape(NC, N, H)

    W2p = jnp.pad(W2, ((0, 0), (0, CP - C)))
    b2p = jnp.pad(b2, (0, CP - C)).reshape(1, CP)
    z2p = _tc_stage2(out1, z1p, dinv, b1.reshape(1, H), W2p)

    out2 = _sc_propagate(z2p, row_r, col_r, CP).reshape(NC, N, CP)
    logits = _tc_stage3(out2, z2p, dinv, b2p)
    return logits


# speedup vs baseline: 1.2726x; 1.0072x over previous
import functools

import jax
import jax.numpy as jnp
from jax import lax
from jax.experimental import pallas as pl
from jax.experimental.pallas import tpu as pltpu
from jax.experimental.pallas import tpu_sc as plsc

N = 10000
E = 320000
D = 128
H = 64
C = 40
CP = 64

NC = 2
NS = 16
NW = NC * NS
EPT = E // NW
CH = 125
NCHUNK = EPT // CH
RPT = N // NS
RING = 8
LAG = 6

_SC_MESH = dict(core_axis_name="c", subcore_axis_name="s", num_cores=NC,
                num_subcores=NS)


NP = 10240
_DROWS = NP // 16
_DPT = _DROWS // NS


def _sc_degree(col_d):
    def body(col_hbm, deg_hbm, colidx_v, hist_v, idx_v, acc_sh):
        cid = lax.axis_index("c")
        sid = lax.axis_index("s")
        wid = cid * NS + sid

        zero16 = jnp.zeros((16,), jnp.float32)

        def zrow(i, _):
            hist_v[i, :] = zero16
            return 0
        lax.fori_loop(0, _DROWS, zrow, 0, unroll=8)

        iota = lax.iota(jnp.int32, 16)

        def irow(c, _):
            def icol(i, _):
                idx_v[c, pl.ds(i * 16, 16)] = iota + (c * 128 + i * 16)
                return 0
            return lax.fori_loop(0, 8, icol, 0)
        lax.fori_loop(0, 5, irow, 0)

        pltpu.sync_copy(hist_v.at[pl.ds(0, _DPT)],
                        acc_sh.at[pl.ds(sid * _DPT, _DPT)])

        pltpu.sync_copy(col_hbm.at[wid], colidx_v)
        plsc.subcore_barrier()

        ones = jnp.ones((16,), jnp.float32)

        def vec(j, _):
            idx = colidx_v[j, :]
            plsc.addupdate_scatter(hist_v, [idx >> 4, idx & 15], ones)
            return 0
        lax.fori_loop(0, EPT // 16, vec, 0, unroll=8)

        for c in range(5):
            pltpu.sync_copy(hist_v.at[pl.ds(c * 128, 128)],
                            acc_sh.at[idx_v.at[c]], add=True)
        plsc.subcore_barrier()
        pltpu.sync_copy(acc_sh.at[pl.ds(sid * _DPT, _DPT)],
                        deg_hbm.at[cid, sid])

    return pl.kernel(
        body,
        out_type=jax.ShapeDtypeStruct((NC, NS, _DPT, 16), jnp.float32),
        mesh=plsc.VectorSubcoreMesh(**_SC_MESH),
        compiler_params=pltpu.CompilerParams(needs_layout_passes=False,
                                             use_tc_tiling_on_sc=False),
        scratch_types=[
            pltpu.VMEM((EPT // 16, 16), jnp.int32),
            pltpu.VMEM((_DROWS, 16), jnp.float32),
            pltpu.VMEM((5, 128), jnp.int32),
            pltpu.VMEM_SHARED((_DROWS, 16), jnp.float32),
        ],
    )(col_d)


def _sc_propagate(z, row_r, col_r, feat):
    def body(z_hbm, row_hbm, col_hbm, out_hbm, rowidx_v, colidx_v, rows_v,
             zbuf_v, acc_sh, gsem, ssem):
        cid = lax.axis_index("c")
        sid = lax.axis_index("s")
        wid = cid * NS + sid

        rowcp = pltpu.async_copy(row_hbm.at[wid], rowidx_v, gsem.at[0])
        colcp = pltpu.async_copy(col_hbm.at[wid], colidx_v, gsem.at[1])

        def zrow(i, _):
            def zcol(j, _):
                zbuf_v[i, pl.ds(j * 32, 32)] = jnp.zeros((32,), jnp.bfloat16)
                return 0
            return lax.fori_loop(0, feat // 32, zcol, 0)
        lax.fori_loop(0, 25, zrow, 0)

        def zcopy(k, _):
            pltpu.sync_copy(zbuf_v, acc_sh.at[pl.ds(sid * RPT + k * 25, 25)])
            return 0
        lax.fori_loop(0, RPT // 25, zcopy, 0)

        rowcp.wait()
        colcp.wait()
        plsc.subcore_barrier()

        def issue_gather(j, b):
            pltpu.async_copy(z_hbm.at[rowidx_v.at[j]], rows_v.at[b],
                             gsem.at[b])

        def wait_gather(j, b):
            pltpu.make_async_copy(z_hbm.at[rowidx_v.at[j]], rows_v.at[b],
                                  gsem.at[b]).wait()

        def issue_scatter(j, b):
            pltpu.async_copy(rows_v.at[b], acc_sh.at[colidx_v.at[j]],
                             ssem.at[b], add=True)

        def wait_scatter(j, b):
            pltpu.make_async_copy(rows_v.at[b], acc_sh.at[colidx_v.at[j]],
                                  ssem.at[b]).wait()

        for j in range(LAG):
            issue_gather(j, j % RING)

        def outer(g, _):
            for b in range(RING):
                j = g * RING + b
                wait_gather(j, b)
                issue_scatter(j, b)
                jg = j + LAG
                bg = (b + LAG) % RING

                @pl.when(jg >= RING)
                def _():
                    wait_scatter(jg - RING, bg)

                @pl.when(jg < NCHUNK)
                def _():
                    issue_gather(jg, bg)
            return 0
        lax.fori_loop(0, NCHUNK // RING, outer, 0)

        for j in range(NCHUNK - (RING - LAG), NCHUNK):
            wait_scatter(j, j % RING)

        plsc.subcore_barrier()
        pltpu.sync_copy(acc_sh.at[pl.ds(sid * RPT, RPT)],
                        out_hbm.at[cid, sid])

    return pl.kernel(
        body,
        out_type=jax.ShapeDtypeStruct((NC, NS, RPT, feat), jnp.bfloat16),
        mesh=plsc.VectorSubcoreMesh(**_SC_MESH),
        compiler_params=pltpu.CompilerParams(needs_layout_passes=False,
                                             use_tc_tiling_on_sc=False),
        scratch_types=[
            pltpu.VMEM((NCHUNK, CH), jnp.int32),
            pltpu.VMEM((NCHUNK, CH), jnp.int32),
            pltpu.VMEM((RING, CH, feat), jnp.bfloat16),
            pltpu.VMEM((25, feat), jnp.bfloat16),
            pltpu.VMEM_SHARED((N, feat), jnp.bfloat16),
            pltpu.SemaphoreType.DMA((RING,)),
            pltpu.SemaphoreType.DMA((RING,)),
        ],
    )(z, row_r, col_r)


_BLK = 1000
_GRID = N // _BLK


def _tc_stage1(x, W1, degp):
    def body(x_ref, w1_ref, p_ref, z1p_ref, dinv_ref):
        deg = 1.0 + p_ref[0] + p_ref[1]
        dinv = lax.rsqrt(deg)
        z = jnp.dot(x_ref[...], w1_ref[...],
                    preferred_element_type=jnp.float32)
        z1p_ref[...] = (z * dinv).astype(jnp.bfloat16)
        dinv_ref[...] = dinv

    return pl.pallas_call(
        body,
        grid=(_GRID,),
        in_specs=[
            pl.BlockSpec((_BLK, D), lambda i: (i, 0)),
            pl.BlockSpec((D, H), lambda i: (0, 0)),
            pl.BlockSpec((NC, _BLK, 1), lambda i: (0, i, 0)),
        ],
        out_specs=[
            pl.BlockSpec((_BLK, H), lambda i: (i, 0)),
            pl.BlockSpec((_BLK, 1), lambda i: (i, 0)),
        ],
        out_shape=[
            jax.ShapeDtypeStruct((N, H), jnp.bfloat16),
            jax.ShapeDtypeStruct((N, 1), jnp.float32),
        ],
    )(x, W1, degp)


def _tc_stage2(out1, z1p, dinv, b1, W2p):
    def body(p_ref, z1p_ref, dinv_ref, b1_ref, w2_ref, z2p_ref):
        dinv = dinv_ref[...]
        s = (p_ref[0].astype(jnp.float32) + p_ref[1].astype(jnp.float32)
             + z1p_ref[...].astype(jnp.float32))
        h = jnp.maximum(s * dinv + b1_ref[...], 0.0)
        z2 = jnp.dot(h, w2_ref[...], preferred_element_type=jnp.float32)
        z2p_ref[...] = (z2 * dinv).astype(jnp.bfloat16)

    return pl.pallas_call(
        body,
        grid=(_GRID,),
        in_specs=[
            pl.BlockSpec((NC, _BLK, H), lambda i: (0, i, 0)),
            pl.BlockSpec((_BLK, H), lambda i: (i, 0)),
            pl.BlockSpec((_BLK, 1), lambda i: (i, 0)),
            pl.BlockSpec((1, H), lambda i: (0, 0)),
            pl.BlockSpec((H, CP), lambda i: (0, 0)),
        ],
        out_specs=pl.BlockSpec((_BLK, CP), lambda i: (i, 0)),
        out_shape=jax.ShapeDtypeStruct((N, CP), jnp.bfloat16),
    )(out1, z1p, dinv, b1, W2p)


def _tc_stage3(out2, z2p, dinv, b2p):
    def body(p_ref, z2p_ref, dinv_ref, b2_ref, out_ref):
        s = (p_ref[0].astype(jnp.float32) + p_ref[1].astype(jnp.float32)
             + z2p_ref[...].astype(jnp.float32))
        full = s * dinv_ref[...] + b2_ref[...]
        out_ref[...] = full[:, :C]

    return pl.pallas_call(
        body,
        grid=(_GRID,),
        in_specs=[
            pl.BlockSpec((NC, _BLK, CP), lambda i: (0, i, 0)),
            pl.BlockSpec((_BLK, CP), lambda i: (i, 0)),
            pl.BlockSpec((_BLK, 1), lambda i: (i, 0)),
            pl.BlockSpec((1, CP), lambda i: (0, 0)),
        ],
        out_specs=pl.BlockSpec((_BLK, C), lambda i: (i, 0)),
        out_shape=jax.ShapeDtypeStruct((N, C), jnp.float32),
    )(out2, z2p, dinv, b2p)


def kernel(x, edge_index, W1, b1, W2, b2):
    ei = edge_index.astype(jnp.int32)
    row_r = ei[0].reshape(NW, NCHUNK, CH)
    col_r = ei[1].reshape(NW, NCHUNK, CH)
    col_d = ei[1].reshape(NW, EPT // 16, 16)

    degp = _sc_degree(col_d).reshape(NC, NP, 1)

    z1p, dinv = _tc_stage1(x, W1, degp)
    out1 = _sc_propagate(z1p, row_r, col_r, H).reshape(NC, N, H)

    W2p = jnp.pad(W2, ((0, 0), (0, CP - C)))
    b2p = jnp.pad(b2, (0, CP - C)).reshape(1, CP)
    z2p = _tc_stage2(out1, z1p, dinv, b1.reshape(1, H), W2p)

    out2 = _sc_propagate(z2p, row_r, col_r, CP).reshape(NC, N, CP)
    logits = _tc_stage3(out2, z2p, dinv, b2p)
    return logits
